# Initial kernel scaffold; baseline (speedup 1.0000x reference)
import jax
import jax.numpy as jnp
from jax.experimental import pallas as pl


def kernel(embedded, centers):
    raise NotImplementedError("write your pallas kernel here")


# speedup vs baseline: 3.6629x; 3.6629x over previous
import jax
import jax.numpy as jnp
from jax.experimental import pallas as pl
from jax.experimental.pallas import tpu as pltpu

_BLOCK = 8192


def _tc_body(emb_ref, cen_ref, out_ref):
    x = emb_ref[...]
    c0 = cen_ref[0, :]
    c1 = cen_ref[1, :]
    d0 = jnp.sum((x - c0[None, :]) ** 2, axis=-1)
    d1 = jnp.sum((x - c1[None, :]) ** 2, axis=-1)
    out_ref[...] = jnp.sqrt(jnp.minimum(d0, d1) + 1e-12)


def kernel(embedded, centers):
    n, d = embedded.shape
    grid = (n // _BLOCK,)
    return pl.pallas_call(
        _tc_body,
        grid=grid,
        in_specs=[
            pl.BlockSpec((_BLOCK, d), lambda i: (i, 0)),
            pl.BlockSpec((2, d), lambda i: (0, 0)),
        ],
        out_specs=pl.BlockSpec((_BLOCK,), lambda i: (i,)),
        out_shape=jax.ShapeDtypeStruct((n,), jnp.float32),
    )(embedded, centers)
